# baseline (speedup 1.0000x reference)
import functools

import numpy as np

import jax
import jax.numpy as jnp
from jax import lax
from jax.experimental import pallas as pl
from jax.experimental.pallas import tpu as pltpu
from jax.experimental.pallas import tpu_sc as plsc

B, S = 4096, 200
VOCAB = 100000
D_MODEL, HIDDEN, N_CLASSES = 64, 128, 1
NUM_CORES, NUM_SUBCORES, LANES = 2, 16, 16
NW = NUM_CORES * NUM_SUBCORES
NSPLIT = 1
B_K = B // NSPLIT
ROWS_PER_TILE = B_K // NW


def _pool_body(x_hbm, table_hbm, out_hbm, idx_v, rows_a, rows_b, rows_c,
               rows_d, out_v, sem_a, sem_b, sem_c, sem_d):
    wid = lax.axis_index("s") * NUM_CORES + lax.axis_index("c")
    base = wid * ROWS_PER_TILE

    g = base // _RPK_BLK
    r = base % _RPK_BLK
    top0 = (2 * _RPK_BLK * g + r) * 128
    bot0 = (2 * _RPK_BLK * g + _RPK_BLK + r) * 128
    pltpu.sync_copy(x_hbm.at[pl.ds(top0, ROWS_PER_TILE * 128)],
                    idx_v.at[pl.ds(0, ROWS_PER_TILE * 128)])
    pltpu.sync_copy(x_hbm.at[pl.ds(bot0, ROWS_PER_TILE * 128)],
                    idx_v.at[pl.ds(ROWS_PER_TILE * 128, ROWS_PER_TILE * 128)])

    def issue(row, buf, sem):
        pltpu.async_copy(
            table_hbm.at[idx_v.at[pl.ds(row * 128, 128)]],
            buf.at[pl.ds(0, 128)],
            sem,
        )
        pltpu.async_copy(
            table_hbm.at[idx_v.at[pl.ds((ROWS_PER_TILE + row) * 128, S - 128)]],
            buf.at[pl.ds(128, S - 128)],
            sem,
        )

    def drain(buf, sem):
        pltpu.make_async_copy(table_hbm.at[pl.ds(0, S)], buf, sem).wait()

    zero_acc = jnp.zeros((LANES,), jnp.float32)

    def accum(buf, i_out):
        himask = jnp.full((LANES,), -65536, jnp.int32)

        def split(w):
            even = plsc.bitcast(w << 16, jnp.float32)
            odd = plsc.bitcast(w & himask, jnp.float32)
            return even, odd

        def acc_body(t, accs):
            a0, b0, a1, b1 = accs
            for k in range(4):
                xa, ya = split(buf[4 * t + k, pl.ds(0, LANES)])
                xb, yb = split(buf[4 * t + k, pl.ds(LANES, LANES)])
                a0 = a0 + xa
                b0 = b0 + ya
                a1 = a1 + xb
                b1 = b1 + yb
            return (a0, b0, a1, b1)

        acc = lax.fori_loop(0, S // 4, acc_body, (zero_acc,) * 4)
        for j in range(4):
            out_v[i_out, pl.ds(j * LANES, LANES)] = acc[j]

    last = ROWS_PER_TILE - 1
    bufs = (rows_a, rows_b, rows_c, rows_d)
    sems = (sem_a, sem_b, sem_c, sem_d)
    for k in range(4):
        issue(k, bufs[k], sems[k])

    def row_quad(g, carry):
        for k in range(4):
            r = 4 * g + k
            drain(bufs[k], sems[k])
            accum(bufs[k], r)
            issue(jnp.minimum(r + 4, last), bufs[k], sems[k])
        return carry

    lax.fori_loop(0, ROWS_PER_TILE // 4, row_quad, 0)
    for k in range(4):
        drain(bufs[k], sems[k])
    pltpu.sync_copy(out_v, out_hbm.at[pl.ds(base, ROWS_PER_TILE)])


_pool = functools.partial(
    pl.kernel,
    mesh=plsc.VectorSubcoreMesh(core_axis_name="c", subcore_axis_name="s"),
    out_type=jax.ShapeDtypeStruct((B_K, D_MODEL), jnp.float32),
    scratch_types=[
        pltpu.VMEM((2 * ROWS_PER_TILE * 128,), jnp.int32),
        pltpu.VMEM((S, D_MODEL // 2), jnp.int32),
        pltpu.VMEM((S, D_MODEL // 2), jnp.int32),
        pltpu.VMEM((S, D_MODEL // 2), jnp.int32),
        pltpu.VMEM((S, D_MODEL // 2), jnp.int32),
        pltpu.VMEM((ROWS_PER_TILE, D_MODEL), jnp.float32),
        pltpu.SemaphoreType.DMA,
        pltpu.SemaphoreType.DMA,
        pltpu.SemaphoreType.DMA,
        pltpu.SemaphoreType.DMA,
    ],
    compiler_params=pltpu.CompilerParams(use_tc_tiling_on_sc=False,
                                         needs_layout_passes=False),
)(_pool_body)


def _mlp_body(s_ref, x_ref, t0_ref, w1_ref, b1_ref, w2_ref, b2_ref,
              wh_ref, bh_ref, o_ref):
    zf = jnp.sum((x_ref[...] == 0).astype(jnp.float32), axis=1, keepdims=True)
    denom = jnp.maximum(jnp.float32(S) - zf, 1.0)
    pooled = (s_ref[...] - zf * t0_ref[...]) / denom
    h1 = jnp.dot(pooled, w1_ref[...], preferred_element_type=jnp.float32)
    h1 = jnp.maximum(h1 + b1_ref[...], 0.0)
    h2 = jnp.dot(h1, w2_ref[...], preferred_element_type=jnp.float32)
    h2 = jnp.maximum(h2 + b2_ref[...], 0.0)
    o_ref[...] = jnp.dot(h2, wh_ref[...], preferred_element_type=jnp.float32) + bh_ref[...]


_RPK_BLK = 512


def _repack_body(x_ref, o_ref):
    m = x_ref[...]
    half = _RPK_BLK * 128
    botp = jnp.concatenate(
        [m[:, 128:], jnp.zeros((_RPK_BLK, 256 - S), jnp.int32)], axis=1
    )
    o_ref[pl.ds(0, half)] = m[:, :128].reshape(half)
    o_ref[pl.ds(half, half)] = botp.reshape(half)


def _repack(x):
    grid = (B // _RPK_BLK,)
    return pl.pallas_call(
        _repack_body,
        grid=grid,
        in_specs=[pl.BlockSpec((_RPK_BLK, S), lambda i: (i, 0))],
        out_specs=pl.BlockSpec((2 * _RPK_BLK * 128,), lambda i: (i,)),
        out_shape=jax.ShapeDtypeStruct((2 * B * 128,), jnp.int32),
    )(x)


_MLP_BLK = 1024


def _mlp(sums, x, table0, W1, b1, W2, b2, Wh, bh):
    grid = (B_K // _MLP_BLK,)
    return pl.pallas_call(
        _mlp_body,
        grid=grid,
        in_specs=[
            pl.BlockSpec((_MLP_BLK, D_MODEL), lambda i: (i, 0)),
            pl.BlockSpec((_MLP_BLK, S), lambda i: (i, 0)),
            pl.BlockSpec((1, D_MODEL), lambda i: (0, 0)),
            pl.BlockSpec((D_MODEL, HIDDEN), lambda i: (0, 0)),
            pl.BlockSpec((1, HIDDEN), lambda i: (0, 0)),
            pl.BlockSpec((HIDDEN, HIDDEN), lambda i: (0, 0)),
            pl.BlockSpec((1, HIDDEN), lambda i: (0, 0)),
            pl.BlockSpec((HIDDEN, N_CLASSES), lambda i: (0, 0)),
            pl.BlockSpec((1, N_CLASSES), lambda i: (0, 0)),
        ],
        out_specs=pl.BlockSpec((_MLP_BLK, N_CLASSES), lambda i: (i, 0)),
        out_shape=jax.ShapeDtypeStruct((B_K, N_CLASSES), jnp.float32),
    )(sums, x, table0, W1, b1.reshape(1, HIDDEN), W2, b2.reshape(1, HIDDEN),
      Wh, bh.reshape(1, N_CLASSES))


_PERM = np.concatenate([
    np.arange(0, 32, 2), np.arange(1, 32, 2),
    np.arange(32, 64, 2), np.arange(33, 64, 2),
]).astype(np.int32)


@jax.jit
def kernel(x, table, W1, b1, W2, b2, Wh, bh):
    t16 = table.astype(jnp.bfloat16)
    t32 = jax.lax.bitcast_convert_type(
        t16.reshape(VOCAB, D_MODEL // 2, 2), jnp.int32)
    t0 = t16[0:1].astype(jnp.float32)[:, _PERM]
    w1p = W1[_PERM, :]
    y = _repack(x)
    sums = _pool(y, t32)
    return _mlp(sums, x, t0, w1p, b1, W2, b2, Wh, bh)


# speedup vs baseline: 1.9344x; 1.9344x over previous
import functools

import numpy as np

import jax
import jax.numpy as jnp
from jax import lax
from jax.experimental import pallas as pl
from jax.experimental.pallas import tpu as pltpu
from jax.experimental.pallas import tpu_sc as plsc

B, S = 4096, 200
VOCAB = 100000
D_MODEL, HIDDEN, N_CLASSES = 64, 128, 1
NUM_CORES, NUM_SUBCORES, LANES = 2, 16, 16
NW = NUM_CORES * NUM_SUBCORES
NSPLIT = 1
B_K = B // NSPLIT
ROWS_PER_TILE = B_K // NW


def _pool_body(x_hbm, table_hbm, out_hbm, idx_v, rows_a, rows_b, rows_c,
               rows_d, out_v, sem_a, sem_b, sem_c, sem_d):
    wid = lax.axis_index("s") * NUM_CORES + lax.axis_index("c")
    base = wid * ROWS_PER_TILE

    g = base // _RPK_BLK
    r = base % _RPK_BLK
    top0 = (2 * _RPK_BLK * g + r) * 128
    bot0 = (2 * _RPK_BLK * g + _RPK_BLK + r) * 128
    pltpu.sync_copy(x_hbm.at[pl.ds(top0, ROWS_PER_TILE * 128)],
                    idx_v.at[pl.ds(0, ROWS_PER_TILE * 128)])
    pltpu.sync_copy(x_hbm.at[pl.ds(bot0, ROWS_PER_TILE * 128)],
                    idx_v.at[pl.ds(ROWS_PER_TILE * 128, ROWS_PER_TILE * 128)])

    def issue(row, buf, sem):
        pltpu.async_copy(
            table_hbm.at[idx_v.at[pl.ds(row * 128, 128)]],
            buf.at[pl.ds(0, 128)],
            sem,
        )
        pltpu.async_copy(
            table_hbm.at[idx_v.at[pl.ds((ROWS_PER_TILE + row) * 128, S - 128)]],
            buf.at[pl.ds(128, S - 128)],
            sem,
        )

    def drain(buf, sem):
        pltpu.make_async_copy(table_hbm.at[pl.ds(0, S)], buf, sem).wait()

    zero_acc = jnp.zeros((LANES,), jnp.float32)

    def accum(buf, i_out):
        himask = jnp.full((LANES,), -65536, jnp.int32)

        def split(w):
            even = plsc.bitcast(w << 16, jnp.float32)
            odd = plsc.bitcast(w & himask, jnp.float32)
            return even, odd

        def acc_body(t, accs):
            a0, b0, a1, b1 = accs
            for k in range(4):
                wa = plsc.bitcast(
                    buf[4 * t + k, pl.ds(0, 2 * LANES)], jnp.int32)
                wb = plsc.bitcast(
                    buf[4 * t + k, pl.ds(2 * LANES, 2 * LANES)], jnp.int32)
                xa, ya = split(wa)
                xb, yb = split(wb)
                a0 = a0 + xa
                b0 = b0 + ya
                a1 = a1 + xb
                b1 = b1 + yb
            return (a0, b0, a1, b1)

        acc = lax.fori_loop(0, S // 4, acc_body, (zero_acc,) * 4)
        for j in range(4):
            out_v[i_out, pl.ds(j * LANES, LANES)] = acc[j]

    last = ROWS_PER_TILE - 1
    bufs = (rows_a, rows_b, rows_c, rows_d)
    sems = (sem_a, sem_b, sem_c, sem_d)
    for k in range(4):
        issue(k, bufs[k], sems[k])

    def row_quad(g, carry):
        for k in range(4):
            r = 4 * g + k
            drain(bufs[k], sems[k])
            accum(bufs[k], r)
            issue(jnp.minimum(r + 4, last), bufs[k], sems[k])
        return carry

    lax.fori_loop(0, ROWS_PER_TILE // 4, row_quad, 0)
    for k in range(4):
        drain(bufs[k], sems[k])
    pltpu.sync_copy(out_v, out_hbm.at[pl.ds(base, ROWS_PER_TILE)])


_pool = functools.partial(
    pl.kernel,
    mesh=plsc.VectorSubcoreMesh(core_axis_name="c", subcore_axis_name="s"),
    out_type=jax.ShapeDtypeStruct((B_K, D_MODEL), jnp.float32),
    scratch_types=[
        pltpu.VMEM((2 * ROWS_PER_TILE * 128,), jnp.int32),
        pltpu.VMEM((S, D_MODEL), jnp.bfloat16),
        pltpu.VMEM((S, D_MODEL), jnp.bfloat16),
        pltpu.VMEM((S, D_MODEL), jnp.bfloat16),
        pltpu.VMEM((S, D_MODEL), jnp.bfloat16),
        pltpu.VMEM((ROWS_PER_TILE, D_MODEL), jnp.float32),
        pltpu.SemaphoreType.DMA,
        pltpu.SemaphoreType.DMA,
        pltpu.SemaphoreType.DMA,
        pltpu.SemaphoreType.DMA,
    ],
    compiler_params=pltpu.CompilerParams(use_tc_tiling_on_sc=False,
                                         needs_layout_passes=False),
)(_pool_body)


def _mlp_body(s_ref, x_ref, t0_ref, w1_ref, b1_ref, w2_ref, b2_ref,
              wh_ref, bh_ref, o_ref):
    zf = jnp.sum((x_ref[...] == 0).astype(jnp.float32), axis=1, keepdims=True)
    denom = jnp.maximum(jnp.float32(S) - zf, 1.0)
    pooled = (s_ref[...] - zf * t0_ref[...]) / denom
    h1 = jnp.dot(pooled, w1_ref[...], preferred_element_type=jnp.float32)
    h1 = jnp.maximum(h1 + b1_ref[...], 0.0)
    h2 = jnp.dot(h1, w2_ref[...], preferred_element_type=jnp.float32)
    h2 = jnp.maximum(h2 + b2_ref[...], 0.0)
    o_ref[...] = jnp.dot(h2, wh_ref[...], preferred_element_type=jnp.float32) + bh_ref[...]


_RPK_BLK = 512


def _repack_body(x_ref, o_ref):
    m = x_ref[...]
    half = _RPK_BLK * 128
    botp = jnp.concatenate(
        [m[:, 128:], jnp.zeros((_RPK_BLK, 256 - S), jnp.int32)], axis=1
    )
    o_ref[pl.ds(0, half)] = m[:, :128].reshape(half)
    o_ref[pl.ds(half, half)] = botp.reshape(half)


def _repack(x):
    grid = (B // _RPK_BLK,)
    return pl.pallas_call(
        _repack_body,
        grid=grid,
        in_specs=[pl.BlockSpec((_RPK_BLK, S), lambda i: (i, 0))],
        out_specs=pl.BlockSpec((2 * _RPK_BLK * 128,), lambda i: (i,)),
        out_shape=jax.ShapeDtypeStruct((2 * B * 128,), jnp.int32),
    )(x)


_TPK_BLK = 2000


def _tpack_body(x_ref, o_ref):
    bits = jax.lax.bitcast_convert_type(x_ref[...], jnp.int32)
    hi = (bits + 0x7FFF + ((bits >> 16) & 1)) >> 16
    lo = hi[:, : D_MODEL // 2]
    up = hi[:, D_MODEL // 2:]
    packed = (lo & 0xFFFF) | (up << 16)
    out128 = jnp.concatenate(
        [packed, jnp.zeros((_TPK_BLK, 96), jnp.int32)], axis=1)
    o_ref[...] = out128.reshape(_TPK_BLK * 128)


def _tpack(table):
    grid = (VOCAB // _TPK_BLK,)
    return pl.pallas_call(
        _tpack_body,
        grid=grid,
        in_specs=[pl.BlockSpec((_TPK_BLK, D_MODEL), lambda i: (i, 0))],
        out_specs=pl.BlockSpec((_TPK_BLK * 128,), lambda i: (i,)),
        out_shape=jax.ShapeDtypeStruct((VOCAB * 128,), jnp.int32),
    )(table)


_MLP_BLK = 1024


def _mlp(sums, x, table0, W1, b1, W2, b2, Wh, bh):
    grid = (B_K // _MLP_BLK,)
    return pl.pallas_call(
        _mlp_body,
        grid=grid,
        in_specs=[
            pl.BlockSpec((_MLP_BLK, D_MODEL), lambda i: (i, 0)),
            pl.BlockSpec((_MLP_BLK, S), lambda i: (i, 0)),
            pl.BlockSpec((1, D_MODEL), lambda i: (0, 0)),
            pl.BlockSpec((D_MODEL, HIDDEN), lambda i: (0, 0)),
            pl.BlockSpec((1, HIDDEN), lambda i: (0, 0)),
            pl.BlockSpec((HIDDEN, HIDDEN), lambda i: (0, 0)),
            pl.BlockSpec((1, HIDDEN), lambda i: (0, 0)),
            pl.BlockSpec((HIDDEN, N_CLASSES), lambda i: (0, 0)),
            pl.BlockSpec((1, N_CLASSES), lambda i: (0, 0)),
        ],
        out_specs=pl.BlockSpec((_MLP_BLK, N_CLASSES), lambda i: (i, 0)),
        out_shape=jax.ShapeDtypeStruct((B_K, N_CLASSES), jnp.float32),
    )(sums, x, table0, W1, b1.reshape(1, HIDDEN), W2, b2.reshape(1, HIDDEN),
      Wh, bh.reshape(1, N_CLASSES))


_PERM = np.concatenate([
    np.arange(0, 32, 2), np.arange(1, 32, 2),
    np.arange(32, 64, 2), np.arange(33, 64, 2),
]).astype(np.int32)


@jax.jit
def kernel(x, table, W1, b1, W2, b2, Wh, bh):
    t16 = table.astype(jnp.bfloat16)
    t0 = t16[0:1].astype(jnp.float32)[:, _PERM]
    w1p = W1[_PERM, :]
    y = _repack(x)
    sums = _pool(y, t16)
    return _mlp(sums, x, t0, w1p, b1, W2, b2, Wh, bh)


# speedup vs baseline: 2.0433x; 1.0563x over previous
import functools

import jax
import jax.numpy as jnp
from jax import lax
from jax.experimental import pallas as pl
from jax.experimental.pallas import tpu as pltpu
from jax.experimental.pallas import tpu_sc as plsc

B, S = 4096, 200
D_MODEL, HIDDEN, N_CLASSES = 64, 128, 1
NUM_CORES, NUM_SUBCORES, LANES = 2, 16, 16
NW = NUM_CORES * NUM_SUBCORES
ROWS_PER_TILE = B // NW


def _pool_body(x_hbm, table_hbm, out_hbm, idx_v, rows_a, rows_b, rows_c,
               rows_d, out_v, sem_a, sem_b, sem_c, sem_d):
    wid = lax.axis_index("s") * NUM_CORES + lax.axis_index("c")
    base = wid * ROWS_PER_TILE

    pltpu.sync_copy(x_hbm.at[pl.ds(base, ROWS_PER_TILE)], idx_v)

    def issue(row, buf, sem):
        pltpu.async_copy(
            table_hbm.at[idx_v.at[row, pl.ds(0, 128)]], buf.at[pl.ds(0, 128)],
            sem,
        )
        pltpu.async_copy(
            table_hbm.at[idx_v.at[row, pl.ds(128, S - 128)]],
            buf.at[pl.ds(128, S - 128)],
            sem,
        )

    def drain(buf, sem):
        pltpu.make_async_copy(table_hbm.at[pl.ds(0, S)], buf, sem).wait()

    zero_acc = jnp.zeros((LANES,), jnp.float32)

    def accum(buf, i_out):
        def acc_body(t, accs):
            a0, a1, a2, a3 = accs
            r = [
                [buf[4 * t + k, pl.ds(j * LANES, LANES)] for j in range(4)]
                for k in range(4)
            ]
            a0 = a0 + ((r[0][0] + r[1][0]) + (r[2][0] + r[3][0]))
            a1 = a1 + ((r[0][1] + r[1][1]) + (r[2][1] + r[3][1]))
            a2 = a2 + ((r[0][2] + r[1][2]) + (r[2][2] + r[3][2]))
            a3 = a3 + ((r[0][3] + r[1][3]) + (r[2][3] + r[3][3]))
            return (a0, a1, a2, a3)

        acc = lax.fori_loop(0, S // 4, acc_body, (zero_acc,) * 4)
        for j in range(4):
            out_v[i_out, pl.ds(j * LANES, LANES)] = acc[j]

    last = ROWS_PER_TILE - 1
    bufs = (rows_a, rows_b, rows_c, rows_d)
    sems = (sem_a, sem_b, sem_c, sem_d)
    for k in range(4):
        issue(k, bufs[k], sems[k])

    def row_quad(g, carry):
        for k in range(4):
            r = 4 * g + k
            drain(bufs[k], sems[k])
            accum(bufs[k], r)
            issue(jnp.minimum(r + 4, last), bufs[k], sems[k])
        return carry

    lax.fori_loop(0, ROWS_PER_TILE // 4, row_quad, 0)
    for k in range(4):
        drain(bufs[k], sems[k])
    pltpu.sync_copy(out_v, out_hbm.at[pl.ds(base, ROWS_PER_TILE)])


_pool = functools.partial(
    pl.kernel,
    mesh=plsc.VectorSubcoreMesh(core_axis_name="c", subcore_axis_name="s"),
    out_type=jax.ShapeDtypeStruct((B, D_MODEL), jnp.float32),
    scratch_types=[
        pltpu.VMEM((ROWS_PER_TILE, S), jnp.int32),
        pltpu.VMEM((S, D_MODEL), jnp.float32),
        pltpu.VMEM((S, D_MODEL), jnp.float32),
        pltpu.VMEM((S, D_MODEL), jnp.float32),
        pltpu.VMEM((S, D_MODEL), jnp.float32),
        pltpu.VMEM((ROWS_PER_TILE, D_MODEL), jnp.float32),
        pltpu.SemaphoreType.DMA,
        pltpu.SemaphoreType.DMA,
        pltpu.SemaphoreType.DMA,
        pltpu.SemaphoreType.DMA,
    ],
    compiler_params=pltpu.CompilerParams(use_tc_tiling_on_sc=False),
)(_pool_body)


def _mlp_body(s_ref, x_ref, t0_ref, w1_ref, b1_ref, w2_ref, b2_ref,
              wh_ref, bh_ref, o_ref):
    zf = jnp.sum((x_ref[...] == 0).astype(jnp.float32), axis=1, keepdims=True)
    denom = jnp.maximum(jnp.float32(S) - zf, 1.0)
    pooled = (s_ref[...] - zf * t0_ref[...]) / denom
    h1 = jnp.dot(pooled, w1_ref[...], preferred_element_type=jnp.float32)
    h1 = jnp.maximum(h1 + b1_ref[...], 0.0)
    h2 = jnp.dot(h1, w2_ref[...], preferred_element_type=jnp.float32)
    h2 = jnp.maximum(h2 + b2_ref[...], 0.0)
    o_ref[...] = jnp.dot(h2, wh_ref[...], preferred_element_type=jnp.float32) + bh_ref[...]


_MLP_BLK = 1024


def _mlp(sums, x, table0, W1, b1, W2, b2, Wh, bh):
    grid = (B // _MLP_BLK,)
    return pl.pallas_call(
        _mlp_body,
        grid=grid,
        in_specs=[
            pl.BlockSpec((_MLP_BLK, D_MODEL), lambda i: (i, 0)),
            pl.BlockSpec((_MLP_BLK, S), lambda i: (i, 0)),
            pl.BlockSpec((1, D_MODEL), lambda i: (0, 0)),
            pl.BlockSpec((D_MODEL, HIDDEN), lambda i: (0, 0)),
            pl.BlockSpec((1, HIDDEN), lambda i: (0, 0)),
            pl.BlockSpec((HIDDEN, HIDDEN), lambda i: (0, 0)),
            pl.BlockSpec((1, HIDDEN), lambda i: (0, 0)),
            pl.BlockSpec((HIDDEN, N_CLASSES), lambda i: (0, 0)),
            pl.BlockSpec((1, N_CLASSES), lambda i: (0, 0)),
        ],
        out_specs=pl.BlockSpec((_MLP_BLK, N_CLASSES), lambda i: (i, 0)),
        out_shape=jax.ShapeDtypeStruct((B, N_CLASSES), jnp.float32),
    )(sums, x, table0, W1, b1.reshape(1, HIDDEN), W2, b2.reshape(1, HIDDEN),
      Wh, bh.reshape(1, N_CLASSES))


@jax.jit
def kernel(x, table, W1, b1, W2, b2, Wh, bh):
    sums = _pool(x, table)
    return _mlp(sums, x, table[0:1], W1, b1, W2, b2, Wh, bh)
